# baseline (speedup 1.0000x reference)
import functools

import jax
import jax.numpy as jnp
from jax import lax
from jax.experimental import pallas as pl

H = 32
PIN = 13
K1 = 64
K2 = 16
N = 10000
NPAD = 10240
POS_INF_KEY = 0x7F800000


def _silu(x):
    return x * jax.nn.sigmoid(x)


def _shift_right(x, s):
    qb = x.shape[0]
    z = jnp.zeros((qb, s), x.dtype)
    return jnp.concatenate([z, x[:, :-s]], axis=1)


def _shift_left(x, s):
    qb = x.shape[0]
    z = jnp.zeros((qb, s), x.dtype)
    return jnp.concatenate([x[:, s:], z], axis=1)


def _inclusive_prefix(x):
    n = x.shape[1]
    s = 1
    while s < n:
        x = x + _shift_right(x, s)
        s *= 2
    return x


def _butterfly_compact(v, sel_i):
    qb, nl = v.shape
    pref = _inclusive_prefix(sel_i)
    lane = lax.broadcasted_iota(jnp.int32, (qb, nl), 1)
    act = sel_i
    dist = (lane - (pref - 1)) * sel_i
    b = 1
    while b < nl:
        bit = ((dist & b) != 0).astype(jnp.int32)
        moving = act * bit
        arr_v = _shift_left(v, b)
        arr_d = _shift_left(dist, b)
        arr_a = _shift_left(act, b)
        arr_bit = ((arr_d & b) != 0).astype(jnp.int32)
        arr_m = (arr_a * arr_bit) != 0
        v = jnp.where(arr_m, arr_v, v)
        dist = jnp.where(arr_m, arr_d - b, dist)
        act = jnp.where(arr_m, 1, act * (1 - moving))
        b *= 2
    return v, pref


def _encode_kernel(xp_ref, xv_ref,
                   w_pfc1, b_pfc1, w_pfc2, b_pfc2,
                   w_v1, b_v1, w_v2, b_v2, w_v3, b_v3,
                   pfc_out, vtx_out):
    xp = xp_ref[...]
    h = _silu(xp @ w_pfc1[...].T + b_pfc1[...])
    pfc_out[...] = h @ w_pfc2[...].T + b_pfc2[...]
    xv = xv_ref[...]
    hv = _silu(xv @ w_v1[...].T + b_v1[...])
    hv = _silu(hv @ w_v2[...].T + b_v2[...])
    vtx_out[...] = hv @ w_v3[...].T + b_v3[...]


def _encode(x_pfc, x_vtx, params):
    p = params
    return pl.pallas_call(
        _encode_kernel,
        out_shape=(jax.ShapeDtypeStruct((x_pfc.shape[0], H), jnp.float32),
                   jax.ShapeDtypeStruct((x_vtx.shape[0], H), jnp.float32)),
    )(x_pfc, x_vtx,
      p["pfc1"]["W"], p["pfc1"]["b"], p["pfc2"]["W"], p["pfc2"]["b"],
      p["vtx1"]["W"], p["vtx1"]["b"], p["vtx2"]["W"], p["vtx2"]["b"],
      p["vtx3"]["W"], p["vtx3"]["b"])


def _head_kernel(x_ref, w1, b1, w2, b2, w3b, out_ref):
    h = _silu(x_ref[...] @ w1[...].T + b1[...])
    h = _silu(h @ w2[...].T + b2[...])
    h8 = jnp.concatenate([h, jnp.ones_like(h)], axis=1)
    out_ref[...] = jnp.sum(h8 * w3b[...], axis=1, keepdims=True)


def _head(x, params):
    p = params
    w3row = p["out3"]["W"].reshape(1, -1)
    pad = jnp.zeros((1, 3), jnp.float32)
    w3b = jnp.concatenate([w3row, p["out3"]["b"].reshape(1, 1), pad], axis=1)
    return pl.pallas_call(
        _head_kernel,
        out_shape=jax.ShapeDtypeStruct((x.shape[0], 1), jnp.float32),
    )(x, p["out1"]["W"], p["out1"]["b"], p["out2"]["W"], p["out2"]["b"], w3b)


def _to_key(bits):
    return bits ^ jnp.where(bits < 0, jnp.int32(0x7FFFFFFF), jnp.int32(0))


def _from_key(key):
    bits = key ^ jnp.where(key < 0, jnp.int32(0x7FFFFFFF), jnp.int32(0))
    return lax.bitcast_convert_type(bits, jnp.float32)


def _knn_thresh_kernel(k, n_col_tiles, max_it,
                       q_ref, post_ref, refsq_ref, sblk_ref,
                       keys_out, t_out, neq_out, cex_out, hc_out):
    qb = q_ref.shape[0]
    npad = keys_out.shape[1]
    ct = npad // n_col_tiles
    q = q_ref[...]
    qsq = jnp.sum(q * q, axis=1, keepdims=True)
    rowmin = jnp.full((qb, 1), POS_INF_KEY, jnp.int32)
    for tt in range(n_col_tiles):
        sl = pl.ds(tt * ct, ct)
        d = qsq - 2.0 * lax.dot_general(
            q, post_ref[:, sl], (((1,), (0,)), ((), ())),
            preferred_element_type=jnp.float32) + refsq_ref[:, sl]
        keys = _to_key(lax.bitcast_convert_type(d, jnp.int32))
        keys_out[:, sl] = keys
        rowmin = jnp.minimum(rowmin, jnp.min(keys, axis=1, keepdims=True))

    lo0 = rowmin - 1
    hi0 = jnp.full((qb, 1), POS_INF_KEY, jnp.int32)
    kk = jnp.int32(k)

    def cond(state):
        it, done_i = state[0], state[1]
        return jnp.logical_and(it < max_it, jnp.min(done_i) < 1)

    def body(state):
        it, done_i, lo, hi, c_lo, c_hi, t, neq = state
        done = done_i != 0
        tie = (lo + 1) == hi
        new_tie = jnp.logical_and(tie, jnp.logical_not(done))
        t = jnp.where(new_tie, hi, t)
        neq = jnp.where(new_tie, kk - c_lo, neq)
        done = jnp.logical_or(done, tie)

        lo_f = _from_key(lo)
        hi_f = _from_key(hi)
        frac = (kk - c_lo).astype(jnp.float32) / jnp.maximum(
            (c_hi - c_lo).astype(jnp.float32), 1.0)
        mid_f = lo_f + (hi_f - lo_f) * frac
        mid_i = _to_key(lax.bitcast_convert_type(mid_f, jnp.int32))
        use_bisect = jnp.logical_or(hi >= POS_INF_KEY, (it % 2) == 1)
        mid_avg = (lo >> 1) + (hi >> 1) + (lo & hi & 1)
        mid = jnp.where(use_bisect, mid_avg, mid_i)
        mid = jnp.clip(mid, lo + 1, hi - 1)

        c = jnp.sum((keys_out[...] <= mid).astype(jnp.int32), axis=1,
                    keepdims=True)
        ge = c >= kk
        upd = jnp.logical_not(done)
        lo = jnp.where(jnp.logical_and(upd, jnp.logical_not(ge)), mid, lo)
        c_lo = jnp.where(jnp.logical_and(upd, jnp.logical_not(ge)), c, c_lo)
        hi = jnp.where(jnp.logical_and(upd, ge), mid, hi)
        c_hi = jnp.where(jnp.logical_and(upd, ge), c, c_hi)
        exact = jnp.logical_and(upd, c == kk)
        t = jnp.where(exact, mid, t)
        neq = jnp.where(exact, kk, neq)
        done = jnp.logical_or(done, exact)
        return (it + 1, done.astype(jnp.int32), lo, hi, c_lo, c_hi, t, neq)

    state = (jnp.int32(0), jnp.zeros((qb, 1), jnp.int32), lo0, hi0,
             jnp.zeros((qb, 1), jnp.int32),
             jnp.full((qb, 1), npad, jnp.int32),
             jnp.full((qb, 1), POS_INF_KEY, jnp.int32),
             jnp.full((qb, 1), k, jnp.int32))
    state = lax.while_loop(cond, body, state)
    _, done_i, lo, hi, c_lo, c_hi, t, neq = state
    done = done_i != 0
    t = jnp.where(done, t, hi)
    neq = jnp.where(done, neq, kk - c_lo)
    t_out[...] = t
    neq_out[...] = neq

    g = npad // 16
    keysv = keys_out[...]
    strict = (keysv < t).astype(jnp.bfloat16)
    eqm = (keysv == t).astype(jnp.bfloat16)
    sblk = sblk_ref[...]
    cs = lax.dot_general(strict, sblk, (((1,), (0,)), ((), ())),
                         preferred_element_type=jnp.float32)
    ce = lax.dot_general(eqm, sblk, (((1,), (0,)), ((), ())),
                         preferred_element_type=jnp.float32)
    gr = lax.broadcasted_iota(jnp.int32, (g, g), 0)
    gc = lax.broadcasted_iota(jnp.int32, (g, g), 1)
    ltri = (gr < gc).astype(jnp.float32)
    ce_excl = lax.dot_general(ce, ltri, (((1,), (0,)), ((), ())),
                              preferred_element_type=jnp.float32,
                              precision=lax.Precision.HIGHEST)
    neqf = neq.astype(jnp.float32)
    kept = jnp.clip(neqf - ce_excl, 0.0, ce)
    scnt = cs + kept
    cex_out[...] = ce_excl.astype(jnp.int32)

    hit = (scnt > 0.5).astype(jnp.int32)
    chunk_id = lax.broadcasted_iota(jnp.int32, (qb, g), 1)
    hc, pref = _butterfly_compact(chunk_id, hit)
    nhit = pref[:, -1:]
    slot = lax.broadcasted_iota(jnp.int32, (qb, g), 1)
    hc = jnp.where(slot < nhit, hc, g - 1)
    hc_out[...] = hc[:, :k]


def _knn_threshold(q, pos_t_pad, refsq_pad, k, qb=80, max_it=40):
    n = q.shape[0]
    npad = pos_t_pad.shape[1]
    g = npad // 16
    sblk = (jnp.arange(npad)[:, None] // 16
            == jnp.arange(g)[None, :]).astype(jnp.bfloat16)
    kern = functools.partial(_knn_thresh_kernel, k, npad // 1024, max_it)
    return pl.pallas_call(
        kern,
        grid=(n // qb,),
        in_specs=[
            pl.BlockSpec((qb, q.shape[1]), lambda i: (i, 0)),
            pl.BlockSpec((pos_t_pad.shape[0], npad), lambda i: (0, 0)),
            pl.BlockSpec((1, npad), lambda i: (0, 0)),
            pl.BlockSpec((npad, g), lambda i: (0, 0)),
        ],
        out_specs=[
            pl.BlockSpec((qb, npad), lambda i: (i, 0)),
            pl.BlockSpec((qb, 1), lambda i: (i, 0)),
            pl.BlockSpec((qb, 1), lambda i: (i, 0)),
            pl.BlockSpec((qb, npad // 16), lambda i: (i, 0)),
            pl.BlockSpec((qb, k), lambda i: (i, 0)),
        ],
        out_shape=[
            jax.ShapeDtypeStruct((n, npad), jnp.int32),
            jax.ShapeDtypeStruct((n, 1), jnp.int32),
            jax.ShapeDtypeStruct((n, 1), jnp.int32),
            jax.ShapeDtypeStruct((n, npad // 16), jnp.int32),
            jax.ShapeDtypeStruct((n, k), jnp.int32),
        ],
    )(q, pos_t_pad, refsq_pad, sblk)


def _extract_kernel(k, ck_ref, ci_ref, cx_ref, t_ref, neq_ref, out_ref):
    qb, kc = ck_ref.shape
    ck = ck_ref[...]
    t = t_ref[...]
    neq = neq_ref[...]
    strict = (ck < t).astype(jnp.int32)
    eq = (ck == t).astype(jnp.int32)
    lane = lax.broadcasted_iota(jnp.int32, (qb, kc), 1)
    lmod = lane & 15
    pe = eq
    for s in (1, 2, 4, 8):
        pe = pe + jnp.where(lmod >= s, _shift_right(pe, s), 0)
    keep = eq * ((cx_ref[...] + pe) <= neq).astype(jnp.int32)
    sel = jnp.maximum(strict, keep)
    v, _ = _butterfly_compact(ci_ref[...], sel)
    out_ref[...] = v[:, :k]


def _extract(cand_keys, cand_idx, cand_cex, t, neq, k, qb=200):
    n, kc = cand_keys.shape
    return pl.pallas_call(
        functools.partial(_extract_kernel, k),
        grid=(n // qb,),
        in_specs=[
            pl.BlockSpec((qb, kc), lambda i: (i, 0)),
            pl.BlockSpec((qb, kc), lambda i: (i, 0)),
            pl.BlockSpec((qb, kc), lambda i: (i, 0)),
            pl.BlockSpec((qb, 1), lambda i: (i, 0)),
            pl.BlockSpec((qb, 1), lambda i: (i, 0)),
        ],
        out_specs=pl.BlockSpec((qb, k), lambda i: (i, 0)),
        out_shape=jax.ShapeDtypeStruct((n, k), jnp.int32),
    )(cand_keys, cand_idx, cand_cex, t, neq)


def _padded_t(x):
    return jnp.pad(x.T, ((0, 0), (0, NPAD - x.shape[0])))


def _knn(qfeat, refsq_pad, k):
    n = qfeat.shape[0]
    keys, t, neq, cex, hc = _knn_threshold(qfeat, _padded_t(qfeat),
                                           refsq_pad, k)
    base = hc * 16
    cand_pos = (base[:, :, None] + jnp.arange(16)).reshape(n, 16 * k)
    cand_keys = jnp.take_along_axis(keys, cand_pos, axis=1)
    cand_cex = jnp.repeat(jnp.take_along_axis(cex, hc, axis=1), 16, axis=1)
    return _extract(cand_keys, cand_pos, cand_cex, t, neq, k)


def _conv1_kernel(qb, pos_ref, g_ref, idx_ref,
                  w_src, w_dst, w_lin, w_p1, b_p1, w_p2, b_p2, d0_ref,
                  out_ref):
    k = idx_ref.shape[1]
    pos = pos_ref[...]
    g3 = g_ref[...]
    g2 = g3.reshape(qb * k, H)
    a_dst = pos @ w_dst[...].T
    a_src_self = pos @ w_src[...].T
    xs_self = pos @ w_lin[...].T
    a_src_g = (g2 @ w_src[...].T).reshape(qb, k, H)
    xs_g = (g2 @ w_lin[...].T).reshape(qb, k, H)

    rel2 = (pos[:, None, :] - g3).reshape(qb * k, H)
    hmid = _silu(rel2 @ w_p1[...].T + b_p1[...])
    delta = (hmid @ w_p2[...].T + b_p2[...]).reshape(qb, k, H)
    d0 = d0_ref[...]

    i0 = pl.program_id(0) * qb + lax.broadcasted_iota(jnp.int32, (qb, 1), 0)
    validf = (idx_ref[...] != i0).astype(jnp.float32)
    v3 = validf[:, :, None]
    alpha = a_dst[:, None, :] - a_src_g + delta
    alpha = alpha * v3 - 1e30 * (1.0 - v3)
    alpha_self = a_dst - a_src_self + d0

    amax = jnp.maximum(jnp.max(alpha, axis=1), alpha_self)
    amax = jnp.where(amax <= -1e29, 0.0, amax)
    aexp = jnp.exp(alpha - amax[:, None, :]) * v3
    aexp_self = jnp.exp(alpha_self - amax)
    denom = jnp.sum(aexp, axis=1) + aexp_self + 1e-16
    msg = jnp.sum(aexp * (xs_g + delta), axis=1)
    msg = msg + aexp_self * (xs_self + d0)
    out_ref[...] = msg / denom


def _conv1(pos, g, idx, params, qb=200):
    p = params
    n = pos.shape[0]
    z = jnp.zeros((1, H), jnp.float32)
    d0 = _silu(z @ p["c1_pos1"]["W"].T + p["c1_pos1"]["b"]) @ \
        p["c1_pos2"]["W"].T + p["c1_pos2"]["b"]
    full = lambda i: (0, 0)
    return pl.pallas_call(
        functools.partial(_conv1_kernel, qb),
        grid=(n // qb,),
        in_specs=[
            pl.BlockSpec((qb, H), lambda i: (i, 0)),
            pl.BlockSpec((qb, K1, H), lambda i: (i, 0, 0)),
            pl.BlockSpec((qb, K1), lambda i: (i, 0)),
            pl.BlockSpec((H, H), full), pl.BlockSpec((H, H), full),
            pl.BlockSpec((H, H), full),
            pl.BlockSpec((H, H), full), pl.BlockSpec((1, H), full),
            pl.BlockSpec((H, H), full), pl.BlockSpec((1, H), full),
            pl.BlockSpec((1, H), full),
        ],
        out_specs=pl.BlockSpec((qb, H), lambda i: (i, 0)),
        out_shape=jax.ShapeDtypeStruct((n, H), jnp.float32),
    )(pos, g, idx,
      p["c1_src"]["W"], p["c1_dst"]["W"], p["c1_lin"]["W"],
      p["c1_pos1"]["W"], p["c1_pos1"]["b"].reshape(1, H),
      p["c1_pos2"]["W"], p["c1_pos2"]["b"].reshape(1, H), d0)


def _conv2_kernel(qb, xp_ref, pd_ref, g_ref, selfg_ref, sv_ref,
                  w_src, w_dst, w_lin, w_p1, b_p1, w_p2, b_p2,
                  out_ref):
    k = g_ref.shape[1]
    xp = xp_ref[...]
    pd = pd_ref[...]
    g3 = g_ref[...]
    feats_j = g3[:, :, :H]
    xpfc_j2 = g3[:, :, H:H + PIN].reshape(qb * k, PIN)
    rank_j = g3[:, :, H + PIN:H + PIN + 1]
    selfg = selfg_ref[...]
    sv = sv_ref[...]

    a_dst = xp @ w_dst[...].T
    a_src_g = (xpfc_j2 @ w_src[...].T).reshape(qb, k, H)
    xs_g = (xpfc_j2 @ w_lin[...].T).reshape(qb, k, H)
    a_src_self = selfg[:, H:H + PIN] @ w_src[...].T
    xs_self = selfg[:, H:H + PIN] @ w_lin[...].T

    rel2 = (pd[:, None, :] - feats_j).reshape(qb * k, H)
    hmid = _silu(rel2 @ w_p1[...].T + b_p1[...])
    delta = (hmid @ w_p2[...].T + b_p2[...]).reshape(qb, k, H)
    rel_self = pd - selfg[:, :H]
    hs = _silu(rel_self @ w_p1[...].T + b_p1[...])
    delta_self = hs @ w_p2[...].T + b_p2[...]

    i0 = pl.program_id(0) * qb + lax.broadcasted_iota(jnp.int32, (qb, 1), 0)
    i0f = i0.astype(jnp.float32)
    validf = (rank_j[:, :, 0] != i0f).astype(jnp.float32)
    v3 = validf[:, :, None]
    alpha = a_dst[:, None, :] - a_src_g + delta
    alpha = alpha * v3 - 1e30 * (1.0 - v3)
    alpha_self = a_dst - a_src_self + delta_self
    alpha_self = alpha_self * sv - 1e30 * (1.0 - sv)

    amax = jnp.maximum(jnp.max(alpha, axis=1), alpha_self)
    amax = jnp.where(amax <= -1e29, 0.0, amax)
    aexp = jnp.exp(alpha - amax[:, None, :]) * v3
    aexp_self = jnp.exp(alpha_self - amax) * sv
    denom = jnp.sum(aexp, axis=1) + aexp_self + 1e-16
    msg = jnp.sum(aexp * (xs_g + delta), axis=1)
    msg = msg + aexp_self * (xs_self + delta_self)
    out_ref[...] = msg / denom


def _conv2(x_pfc, feats1, g, selfg, self_valid, params, qb=200):
    p = params
    n = x_pfc.shape[0]
    dt = g.shape[2]
    full = lambda i: (0, 0)
    return pl.pallas_call(
        functools.partial(_conv2_kernel, qb),
        grid=(n // qb,),
        in_specs=[
            pl.BlockSpec((qb, PIN), lambda i: (i, 0)),
            pl.BlockSpec((qb, H), lambda i: (i, 0)),
            pl.BlockSpec((qb, K2, dt), lambda i: (i, 0, 0)),
            pl.BlockSpec((qb, dt), lambda i: (i, 0)),
            pl.BlockSpec((qb, 1), lambda i: (i, 0)),
            pl.BlockSpec((H, PIN), full), pl.BlockSpec((H, PIN), full),
            pl.BlockSpec((H, PIN), full),
            pl.BlockSpec((H, H), full), pl.BlockSpec((1, H), full),
            pl.BlockSpec((H, H), full), pl.BlockSpec((1, H), full),
        ],
        out_specs=pl.BlockSpec((qb, H), lambda i: (i, 0)),
        out_shape=jax.ShapeDtypeStruct((n, H), jnp.float32),
    )(x_pfc, feats1, g, selfg, self_valid,
      p["c2_src"]["W"], p["c2_dst"]["W"], p["c2_lin"]["W"],
      p["c2_pos1"]["W"], p["c2_pos1"]["b"].reshape(1, H),
      p["c2_pos2"]["W"], p["c2_pos2"]["b"].reshape(1, H))


def kernel(x_pfc, x_vtx, batch_pfc, batch_vtx, params):
    n = x_pfc.shape[0]
    inf = jnp.float32(jnp.inf)

    x_pfc_enc, x_vtx_enc = _encode(x_pfc, x_vtx, params)
    pos = x_pfc_enc

    refsq1 = jnp.sum(pos * pos, axis=1)
    refsq1_pad = jnp.pad(refsq1, (0, NPAD - n),
                         constant_values=inf).reshape(1, NPAD)
    idx1 = _knn(pos, refsq1_pad, K1)
    g1 = pos[idx1]
    feats1 = _conv1(pos, g1, idx1, params)

    charged = x_pfc[:, -2] != 0
    count = jnp.sum(charged.astype(jnp.int32))
    rank = jnp.cumsum(charged.astype(jnp.int32)) - 1
    perm = jnp.argsort(jnp.where(charged, 0, 1))

    refsq2 = jnp.where(charged, jnp.sum(feats1 * feats1, axis=1), inf)
    refsq2_pad = jnp.pad(refsq2, (0, NPAD - n),
                         constant_values=inf).reshape(1, NPAD)
    idx2 = _knn(feats1, refsq2_pad, K2)
    table2 = jnp.concatenate(
        [feats1, x_pfc, rank.astype(jnp.float32)[:, None],
         jnp.zeros((n, 2), jnp.float32)], axis=1)
    g2 = table2[idx2]
    selfg2 = table2[perm]
    self_valid = (jnp.arange(n) < count).astype(jnp.float32)[:, None]
    feats2 = _conv2(x_pfc, feats1, g2, selfg2, self_valid, params)

    out = _head(feats2, params)
    return (out, batch_pfc, feats1, x_vtx_enc)
